# baseline (speedup 1.0000x reference)
import functools

import jax
import jax.numpy as jnp
from jax import lax
from jax.experimental import pallas as pl
from jax.experimental.pallas import tpu as pltpu
from jax.experimental.pallas import tpu_sc as plsc

_L = 16
_NW = 32
_E = 1024
_ENT = 32
_PAIRS = 16
_BB = 64


def _pack_table(t):
    v = t.shape[0]
    b = t.astype(jnp.bfloat16).reshape(v, _PAIRS, 2)
    return lax.bitcast_convert_type(b, jnp.int32).reshape(-1)


def _sc_embed(p0_i, p1_i, p2_i, tables, bases):
    n = p0_i.shape[0]
    per_w = n // _NW
    n_chunk = per_w // _E
    mesh = plsc.VectorSubcoreMesh(core_axis_name="c", subcore_axis_name="s")

    @functools.partial(
        pl.kernel,
        out_type=jax.ShapeDtypeStruct((n * _PAIRS,), jnp.int32),
        mesh=mesh,
        compiler_params=pltpu.CompilerParams(needs_layout_passes=False),
        scratch_types=[
            pltpu.VMEM(tables.shape, jnp.int32),
            pltpu.VMEM((_E,), jnp.int32),
            pltpu.VMEM((_E,), jnp.int32),
            pltpu.VMEM((_E,), jnp.int32),
            pltpu.VMEM((_E * _PAIRS,), jnp.int32),
        ],
    )
    def k(p0_h, p1_h, p2_h, tbl_h, out, t_all, b_p0, b_p1, b_p2, b_emb):
        wid = lax.axis_index("s") * 2 + lax.axis_index("c")
        pltpu.sync_copy(tbl_h, t_all)
        base_w = wid * per_w

        @pl.loop(0, n_chunk)
        def _chunk(c):
            base = base_w + c * _E
            pltpu.sync_copy(p0_h.at[pl.ds(base, _E)], b_p0)
            pltpu.sync_copy(p1_h.at[pl.ds(base, _E)], b_p1)
            pltpu.sync_copy(p2_h.at[pl.ds(base, _E)], b_p2)

            @plsc.parallel_loop(0, _E, _L, unroll=4)
            def _group(s):
                lanes = lax.iota(jnp.int32, _L)
                w0 = b_p0[pl.ds(s, _L)]
                w1 = b_p1[pl.ds(s, _L)]
                w2 = b_p2[pl.ds(s, _L)]
                addr = [
                    (w0 & 0x7FF) * _PAIRS + bases[0],
                    ((w0 >> 11) & 0x1FF) * _PAIRS + bases[1],
                    (w0 >> 20) * _PAIRS + bases[2],
                    (w2 >> 10) * _PAIRS + bases[3],
                    (w1 & 0x3FF) * _PAIRS + bases[4],
                    ((w1 >> 10) & 0x3FF) * _PAIRS + bases[4],
                    (w1 >> 20) * _PAIRS + bases[4],
                    (w2 & 0x3FF) * _PAIRS + bases[4],
                ]
                accs = []
                for pc in range(_PAIRS):
                    acc = plsc.bitcast(
                        plsc.load_gather(t_all, [addr[0] + pc]), jnp.bfloat16)
                    for j in range(1, 8):
                        w = plsc.bitcast(
                            plsc.load_gather(t_all, [addr[j] + pc]),
                            jnp.bfloat16)
                        acc = acc + w
                    accs.append(plsc.bitcast(acc, jnp.int32))
                rows = (lanes + s) * _PAIRS
                for pc in range(_PAIRS):
                    plsc.store_scatter(b_emb, [rows + pc], accs[pc])

            pltpu.sync_copy(b_emb, out.at[pl.ds(base * _PAIRS, _E * _PAIRS)])

    return k(p0_i, p1_i, p2_i, tables)


def _tc_body(emb, bias, hp, hp_w, units_w, units_b, tv_w1, tv_b1,
             tv_w2, tv_b2, out):
    w = emb[...]
    lo = lax.bitcast_convert_type(w << 16, jnp.float32)
    hi = lax.bitcast_convert_type(w & jnp.int32(-65536), jnp.float32)
    x = jnp.concatenate([lo, hi], axis=1)
    x = x + bias[...] + hp[...] * hp_w[...]
    x = jnp.maximum(x, 0.0)
    u = jnp.dot(x, units_w[...], preferred_element_type=jnp.float32)
    u = jnp.maximum(u + units_b[...], 0.0)
    rows = _BB * 18
    grp = lax.broadcasted_iota(jnp.int32, (_BB, rows), 0)
    row = lax.broadcasted_iota(jnp.int32, (_BB, rows), 1)
    sel = (row // 18 == grp).astype(jnp.float32)
    pooled = jnp.dot(sel, u, preferred_element_type=jnp.float32) * (1.0 / 18.0)
    h = jnp.dot(pooled, tv_w1[...], preferred_element_type=jnp.float32)
    h = jnp.maximum(h + tv_b1[...], 0.0)
    out[...] = (jnp.dot(h, tv_w2[...], preferred_element_type=jnp.float32)
                + tv_b2[...])


def _tc_dense(emb_i32, bias_blk, hp2d, hp_w, units_w, units_b,
              tv_w1, tv_b1, tv_w2, tv_b2, batch):
    rows = _BB * 18
    grid = batch // _BB
    return pl.pallas_call(
        _tc_body,
        grid=(grid,),
        in_specs=[
            pl.BlockSpec((rows, _PAIRS), lambda i: (i, 0)),
            pl.BlockSpec((rows, _ENT), lambda i: (0, 0)),
            pl.BlockSpec((rows, 1), lambda i: (i, 0)),
            pl.BlockSpec((1, _ENT), lambda i: (0, 0)),
            pl.BlockSpec(units_w.shape, lambda i: (0, 0)),
            pl.BlockSpec((1, _ENT), lambda i: (0, 0)),
            pl.BlockSpec(tv_w1.shape, lambda i: (0, 0)),
            pl.BlockSpec((1, 128), lambda i: (0, 0)),
            pl.BlockSpec(tv_w2.shape, lambda i: (0, 0)),
            pl.BlockSpec((1, 128), lambda i: (0, 0)),
        ],
        out_specs=pl.BlockSpec((_BB, 128), lambda i: (i, 0)),
        out_shape=jax.ShapeDtypeStruct((batch, 128), jnp.float32),
    )(emb_i32, bias_blk, hp2d, hp_w, units_w, units_b,
      tv_w1, tv_b1, tv_w2, tv_b2)


def kernel(species, items, abilities, moves, hp_bucket, hp, status, active,
           fainted, species_table, item_table, ability_table, moves_table,
           hp_table, status_table, active_table, fainted_table, side_table,
           public_table, hp_w, hp_b, units_w, units_b, tv_w1, tv_b1, tv_w2,
           tv_b2):
    batch = species.shape[0]
    n = batch * 18

    sm_t = (hp_table[:, None, None, None, :]
            + status_table[None, :, None, None, :]
            + active_table[None, None, :, None, :]
            + fainted_table[None, None, None, :, :]).reshape(-1, _ENT)

    packed = [_pack_table(t) for t in
              (species_table, item_table, ability_table, sm_t, moves_table)]
    off = [0]
    for p in packed[:-1]:
        off.append(off[-1] + p.shape[0])
    bases = (off[0], off[1], off[2], off[3], off[4])
    tables = jnp.concatenate(packed)

    sp = species.reshape(n).astype(jnp.int32)
    it = items.reshape(n).astype(jnp.int32)
    ab = abilities.reshape(n).astype(jnp.int32)
    sm = (((hp_bucket.reshape(n) * 8 + status.reshape(n)) * 4
           + active.reshape(n) * 2 + fainted.reshape(n))
          .astype(jnp.int32))
    mv = moves.reshape(n, 4).astype(jnp.int32)
    p0 = sp | (it << 11) | (ab << 20)
    p1 = mv[:, 0] | (mv[:, 1] << 10) | (mv[:, 2] << 20)
    p2 = mv[:, 3] | (sm << 10)

    emb_i32 = _sc_embed(p0, p1, p2, tables, bases).reshape(n, _PAIRS)

    perm = jnp.arange(_ENT).reshape(_PAIRS, 2).T.reshape(-1)

    side_token = jnp.zeros((3, 6), dtype=jnp.int32).at[-1].set(1)
    public_token = jnp.zeros((3, 6), dtype=jnp.int32).at[1:].set(1)
    pos18 = (jnp.take(side_table, side_token.reshape(-1), axis=0)
             + jnp.take(public_table, public_token.reshape(-1), axis=0)
             + hp_b[None, :])[:, perm]
    bias_blk = jnp.tile(pos18, (_BB, 1))

    out = _tc_dense(emb_i32, bias_blk, hp.reshape(n, 1),
                    hp_w.reshape(1, _ENT)[:, perm], units_w[perm, :],
                    units_b.reshape(1, _ENT), tv_w1,
                    tv_b1.reshape(1, 128), tv_w2, tv_b2.reshape(1, 128),
                    batch)
    return out


# speedup vs baseline: 1.0408x; 1.0408x over previous
import functools

import jax
import jax.numpy as jnp
from jax import lax
from jax.experimental import pallas as pl
from jax.experimental.pallas import tpu as pltpu
from jax.experimental.pallas import tpu_sc as plsc

_L = 16
_NW = 32
_E = 1024
_ENT = 32
_PAIRS = 16
_BB = 64


def _pack_table(t):
    v = t.shape[0]
    b = t.astype(jnp.bfloat16).reshape(v, _PAIRS, 2)
    return lax.bitcast_convert_type(b, jnp.int32).reshape(-1)


def _sc_embed(p0_i, p1_i, p2_i, tables, bases):
    n = p0_i.shape[0]
    per_w = n // _NW
    n_chunk = per_w // _E
    mesh = plsc.VectorSubcoreMesh(core_axis_name="c", subcore_axis_name="s")

    @functools.partial(
        pl.kernel,
        out_type=jax.ShapeDtypeStruct((n * _PAIRS,), jnp.int32),
        mesh=mesh,
        compiler_params=pltpu.CompilerParams(needs_layout_passes=False),
        scratch_types=[
            pltpu.VMEM(tables.shape, jnp.int32),
            pltpu.VMEM((_E,), jnp.int32),
            pltpu.VMEM((_E,), jnp.int32),
            pltpu.VMEM((_E,), jnp.int32),
            pltpu.VMEM((_E * _PAIRS,), jnp.int32),
        ],
    )
    def k(p0_h, p1_h, p2_h, tbl_h, out, t_all, b_p0, b_p1, b_p2, b_emb):
        wid = lax.axis_index("s") * 2 + lax.axis_index("c")
        pltpu.sync_copy(tbl_h, t_all)
        base_w = wid * per_w

        @pl.loop(0, n_chunk)
        def _chunk(c):
            base = base_w + c * _E
            pltpu.sync_copy(p0_h.at[pl.ds(base, _E)], b_p0)
            pltpu.sync_copy(p1_h.at[pl.ds(base, _E)], b_p1)
            pltpu.sync_copy(p2_h.at[pl.ds(base, _E)], b_p2)

            @plsc.parallel_loop(0, _E, _L, unroll=2)
            def _group(s):
                lanes = lax.iota(jnp.int32, _L)
                w0 = b_p0[pl.ds(s, _L)]
                w1 = b_p1[pl.ds(s, _L)]
                w2 = b_p2[pl.ds(s, _L)]
                addr = [
                    (w0 & 0x7FF) * _PAIRS + bases[0],
                    ((w0 >> 11) & 0x1FF) * _PAIRS + bases[1],
                    (w0 >> 20) * _PAIRS + bases[2],
                    (w2 >> 10) * _PAIRS + bases[3],
                    (w1 & 0x3FF) * _PAIRS + bases[4],
                    ((w1 >> 10) & 0x3FF) * _PAIRS + bases[4],
                    (w1 >> 20) * _PAIRS + bases[4],
                    (w2 & 0x3FF) * _PAIRS + bases[4],
                ]
                accs = []
                for pc in range(_PAIRS):
                    acc = plsc.bitcast(
                        plsc.load_gather(t_all, [addr[0] + pc]), jnp.bfloat16)
                    for j in range(1, 8):
                        w = plsc.bitcast(
                            plsc.load_gather(t_all, [addr[j] + pc]),
                            jnp.bfloat16)
                        acc = acc + w
                    accs.append(plsc.bitcast(acc, jnp.int32))
                rows = (lanes + s) * _PAIRS
                for pc in range(_PAIRS):
                    plsc.store_scatter(b_emb, [rows + pc], accs[pc])

            pltpu.sync_copy(b_emb, out.at[pl.ds(base * _PAIRS, _E * _PAIRS)])

    return k(p0_i, p1_i, p2_i, tables)


def _tc_body(emb, bias, hp, hp_w, units_w, units_b, tv_w1, tv_b1,
             tv_w2, tv_b2, out):
    w = emb[...]
    lo = lax.bitcast_convert_type(w << 16, jnp.float32)
    hi = lax.bitcast_convert_type(w & jnp.int32(-65536), jnp.float32)
    x = jnp.concatenate([lo, hi], axis=1)
    x = x + bias[...] + hp[...] * hp_w[...]
    x = jnp.maximum(x, 0.0)
    u = jnp.dot(x, units_w[...], preferred_element_type=jnp.float32)
    u = jnp.maximum(u + units_b[...], 0.0)
    rows = _BB * 18
    grp = lax.broadcasted_iota(jnp.int32, (_BB, rows), 0)
    row = lax.broadcasted_iota(jnp.int32, (_BB, rows), 1)
    sel = (row // 18 == grp).astype(jnp.float32)
    pooled = jnp.dot(sel, u, preferred_element_type=jnp.float32) * (1.0 / 18.0)
    h = jnp.dot(pooled, tv_w1[...], preferred_element_type=jnp.float32)
    h = jnp.maximum(h + tv_b1[...], 0.0)
    out[...] = (jnp.dot(h, tv_w2[...], preferred_element_type=jnp.float32)
                + tv_b2[...])


def _tc_dense(emb_i32, bias_blk, hp2d, hp_w, units_w, units_b,
              tv_w1, tv_b1, tv_w2, tv_b2, batch):
    rows = _BB * 18
    grid = batch // _BB
    return pl.pallas_call(
        _tc_body,
        grid=(grid,),
        in_specs=[
            pl.BlockSpec((rows, _PAIRS), lambda i: (i, 0)),
            pl.BlockSpec((rows, _ENT), lambda i: (0, 0)),
            pl.BlockSpec((rows, 1), lambda i: (i, 0)),
            pl.BlockSpec((1, _ENT), lambda i: (0, 0)),
            pl.BlockSpec(units_w.shape, lambda i: (0, 0)),
            pl.BlockSpec((1, _ENT), lambda i: (0, 0)),
            pl.BlockSpec(tv_w1.shape, lambda i: (0, 0)),
            pl.BlockSpec((1, 128), lambda i: (0, 0)),
            pl.BlockSpec(tv_w2.shape, lambda i: (0, 0)),
            pl.BlockSpec((1, 128), lambda i: (0, 0)),
        ],
        out_specs=pl.BlockSpec((_BB, 128), lambda i: (i, 0)),
        out_shape=jax.ShapeDtypeStruct((batch, 128), jnp.float32),
    )(emb_i32, bias_blk, hp2d, hp_w, units_w, units_b,
      tv_w1, tv_b1, tv_w2, tv_b2)


def kernel(species, items, abilities, moves, hp_bucket, hp, status, active,
           fainted, species_table, item_table, ability_table, moves_table,
           hp_table, status_table, active_table, fainted_table, side_table,
           public_table, hp_w, hp_b, units_w, units_b, tv_w1, tv_b1, tv_w2,
           tv_b2):
    batch = species.shape[0]
    n = batch * 18

    sm_t = (hp_table[:, None, None, None, :]
            + status_table[None, :, None, None, :]
            + active_table[None, None, :, None, :]
            + fainted_table[None, None, None, :, :]).reshape(-1, _ENT)

    packed = [_pack_table(t) for t in
              (species_table, item_table, ability_table, sm_t, moves_table)]
    off = [0]
    for p in packed[:-1]:
        off.append(off[-1] + p.shape[0])
    bases = (off[0], off[1], off[2], off[3], off[4])
    tables = jnp.concatenate(packed)

    sp = species.reshape(n).astype(jnp.int32)
    it = items.reshape(n).astype(jnp.int32)
    ab = abilities.reshape(n).astype(jnp.int32)
    sm = (((hp_bucket.reshape(n) * 8 + status.reshape(n)) * 4
           + active.reshape(n) * 2 + fainted.reshape(n))
          .astype(jnp.int32))
    mv = moves.reshape(n, 4).astype(jnp.int32)
    p0 = sp | (it << 11) | (ab << 20)
    p1 = mv[:, 0] | (mv[:, 1] << 10) | (mv[:, 2] << 20)
    p2 = mv[:, 3] | (sm << 10)

    emb_i32 = _sc_embed(p0, p1, p2, tables, bases).reshape(n, _PAIRS)

    perm = jnp.arange(_ENT).reshape(_PAIRS, 2).T.reshape(-1)

    side_token = jnp.zeros((3, 6), dtype=jnp.int32).at[-1].set(1)
    public_token = jnp.zeros((3, 6), dtype=jnp.int32).at[1:].set(1)
    pos18 = (jnp.take(side_table, side_token.reshape(-1), axis=0)
             + jnp.take(public_table, public_token.reshape(-1), axis=0)
             + hp_b[None, :])[:, perm]
    bias_blk = jnp.tile(pos18, (_BB, 1))

    out = _tc_dense(emb_i32, bias_blk, hp.reshape(n, 1),
                    hp_w.reshape(1, _ENT)[:, perm], units_w[perm, :],
                    units_b.reshape(1, _ENT), tv_w1,
                    tv_b1.reshape(1, 128), tv_w2, tv_b2.reshape(1, 128),
                    batch)
    return out


# speedup vs baseline: 1.0608x; 1.0192x over previous
import functools

import jax
import jax.numpy as jnp
from jax import lax
from jax.experimental import pallas as pl
from jax.experimental.pallas import tpu as pltpu
from jax.experimental.pallas import tpu_sc as plsc

_L = 16
_NW = 32
_E = 1024
_ENT = 32
_PAIRS = 16
_BB = 64


def _pack_table(t):
    v = t.shape[0]
    b = t.astype(jnp.bfloat16).reshape(v, _PAIRS, 2)
    return lax.bitcast_convert_type(b, jnp.int32).reshape(-1)


def _sc_embed(p0_i, p1_i, p2_i, tables, bases, chunk_e=_E):
    _E = chunk_e
    n = p0_i.shape[0]
    per_w = n // _NW
    n_chunk = per_w // _E
    mesh = plsc.VectorSubcoreMesh(core_axis_name="c", subcore_axis_name="s")

    @functools.partial(
        pl.kernel,
        out_type=jax.ShapeDtypeStruct((n * _PAIRS,), jnp.int32),
        mesh=mesh,
        compiler_params=pltpu.CompilerParams(needs_layout_passes=False),
        scratch_types=[
            pltpu.VMEM(tables.shape, jnp.int32),
            pltpu.VMEM((_E,), jnp.int32),
            pltpu.VMEM((_E,), jnp.int32),
            pltpu.VMEM((_E,), jnp.int32),
            pltpu.VMEM((_E * _PAIRS,), jnp.int32),
        ],
    )
    def k(p0_h, p1_h, p2_h, tbl_h, out, t_all, b_p0, b_p1, b_p2, b_emb):
        wid = lax.axis_index("s") * 2 + lax.axis_index("c")
        pltpu.sync_copy(tbl_h, t_all)
        base_w = wid * per_w

        @pl.loop(0, n_chunk)
        def _chunk(c):
            base = base_w + c * _E
            pltpu.sync_copy(p0_h.at[pl.ds(base, _E)], b_p0)
            pltpu.sync_copy(p1_h.at[pl.ds(base, _E)], b_p1)
            pltpu.sync_copy(p2_h.at[pl.ds(base, _E)], b_p2)

            @plsc.parallel_loop(0, _E, _L, unroll=2)
            def _group(s):
                lanes = lax.iota(jnp.int32, _L)
                w0 = b_p0[pl.ds(s, _L)]
                w1 = b_p1[pl.ds(s, _L)]
                w2 = b_p2[pl.ds(s, _L)]
                addr = [
                    (w0 & 0x7FF) * _PAIRS + bases[0],
                    ((w0 >> 11) & 0x1FF) * _PAIRS + bases[1],
                    (w0 >> 20) * _PAIRS + bases[2],
                    (w2 >> 10) * _PAIRS + bases[3],
                    (w1 & 0x3FF) * _PAIRS + bases[4],
                    ((w1 >> 10) & 0x3FF) * _PAIRS + bases[4],
                    (w1 >> 20) * _PAIRS + bases[4],
                    (w2 & 0x3FF) * _PAIRS + bases[4],
                ]
                accs = []
                for pc in range(_PAIRS):
                    acc = plsc.bitcast(
                        plsc.load_gather(t_all, [addr[0] + pc]), jnp.bfloat16)
                    for j in range(1, 8):
                        w = plsc.bitcast(
                            plsc.load_gather(t_all, [addr[j] + pc]),
                            jnp.bfloat16)
                        acc = acc + w
                    accs.append(plsc.bitcast(acc, jnp.int32))
                rows = (lanes + s) * _PAIRS
                for pc in range(_PAIRS):
                    plsc.store_scatter(b_emb, [rows + pc], accs[pc])

            pltpu.sync_copy(b_emb, out.at[pl.ds(base * _PAIRS, _E * _PAIRS)])

    return k(p0_i, p1_i, p2_i, tables)


def _tc_body(emb, bias, hp, hp_w, units_w, units_b, tv_w1, tv_b1,
             tv_w2, tv_b2, out):
    w = emb[...]
    lo = lax.bitcast_convert_type(w << 16, jnp.float32)
    hi = lax.bitcast_convert_type(w & jnp.int32(-65536), jnp.float32)
    x = jnp.concatenate([lo, hi], axis=1)
    x = x + bias[...] + hp[...] * hp_w[...]
    x = jnp.maximum(x, 0.0)
    u = jnp.dot(x, units_w[...], preferred_element_type=jnp.float32)
    u = jnp.maximum(u + units_b[...], 0.0)
    rows = _BB * 18
    grp = lax.broadcasted_iota(jnp.int32, (_BB, rows), 0)
    row = lax.broadcasted_iota(jnp.int32, (_BB, rows), 1)
    sel = (row // 18 == grp).astype(jnp.float32)
    pooled = jnp.dot(sel, u, preferred_element_type=jnp.float32) * (1.0 / 18.0)
    h = jnp.dot(pooled, tv_w1[...], preferred_element_type=jnp.float32)
    h = jnp.maximum(h + tv_b1[...], 0.0)
    out[...] = (jnp.dot(h, tv_w2[...], preferred_element_type=jnp.float32)
                + tv_b2[...])


def _tc_dense(emb_i32, bias_blk, hp2d, hp_w, units_w, units_b,
              tv_w1, tv_b1, tv_w2, tv_b2, batch):
    rows = _BB * 18
    grid = batch // _BB
    return pl.pallas_call(
        _tc_body,
        grid=(grid,),
        in_specs=[
            pl.BlockSpec((rows, _PAIRS), lambda i: (i, 0)),
            pl.BlockSpec((rows, _ENT), lambda i: (0, 0)),
            pl.BlockSpec((rows, 1), lambda i: (i, 0)),
            pl.BlockSpec((1, _ENT), lambda i: (0, 0)),
            pl.BlockSpec(units_w.shape, lambda i: (0, 0)),
            pl.BlockSpec((1, _ENT), lambda i: (0, 0)),
            pl.BlockSpec(tv_w1.shape, lambda i: (0, 0)),
            pl.BlockSpec((1, 128), lambda i: (0, 0)),
            pl.BlockSpec(tv_w2.shape, lambda i: (0, 0)),
            pl.BlockSpec((1, 128), lambda i: (0, 0)),
        ],
        out_specs=pl.BlockSpec((_BB, 128), lambda i: (i, 0)),
        out_shape=jax.ShapeDtypeStruct((batch, 128), jnp.float32),
    )(emb_i32, bias_blk, hp2d, hp_w, units_w, units_b,
      tv_w1, tv_b1, tv_w2, tv_b2)


def kernel(species, items, abilities, moves, hp_bucket, hp, status, active,
           fainted, species_table, item_table, ability_table, moves_table,
           hp_table, status_table, active_table, fainted_table, side_table,
           public_table, hp_w, hp_b, units_w, units_b, tv_w1, tv_b1, tv_w2,
           tv_b2):
    batch = species.shape[0]
    n = batch * 18

    sm_t = (hp_table[:, None, None, None, :]
            + status_table[None, :, None, None, :]
            + active_table[None, None, :, None, :]
            + fainted_table[None, None, None, :, :]).reshape(-1, _ENT)

    packed = [_pack_table(t) for t in
              (species_table, item_table, ability_table, sm_t, moves_table)]
    off = [0]
    for p in packed[:-1]:
        off.append(off[-1] + p.shape[0])
    bases = (off[0], off[1], off[2], off[3], off[4])
    tables = jnp.concatenate(packed)

    sp = species.reshape(n).astype(jnp.int32)
    it = items.reshape(n).astype(jnp.int32)
    ab = abilities.reshape(n).astype(jnp.int32)
    sm = (((hp_bucket.reshape(n) * 8 + status.reshape(n)) * 4
           + active.reshape(n) * 2 + fainted.reshape(n))
          .astype(jnp.int32))
    mv = moves.reshape(n, 4).astype(jnp.int32)
    p0 = sp | (it << 11) | (ab << 20)
    p1 = mv[:, 0] | (mv[:, 1] << 10) | (mv[:, 2] << 20)
    p2 = mv[:, 3] | (sm << 10)

    half = n // 2
    emb_a = _sc_embed(p0[:half], p1[:half], p2[:half], tables, bases,
                      chunk_e=576).reshape(half, _PAIRS)
    emb_b = _sc_embed(p0[half:], p1[half:], p2[half:], tables, bases,
                      chunk_e=576).reshape(half, _PAIRS)

    perm = jnp.arange(_ENT).reshape(_PAIRS, 2).T.reshape(-1)

    side_token = jnp.zeros((3, 6), dtype=jnp.int32).at[-1].set(1)
    public_token = jnp.zeros((3, 6), dtype=jnp.int32).at[1:].set(1)
    pos18 = (jnp.take(side_table, side_token.reshape(-1), axis=0)
             + jnp.take(public_table, public_token.reshape(-1), axis=0)
             + hp_b[None, :])[:, perm]
    bias_blk = jnp.tile(pos18, (_BB, 1))

    hp2d = hp.reshape(n, 1)
    hpw_p = hp_w.reshape(1, _ENT)[:, perm]
    uw_p = units_w[perm, :]
    ub = units_b.reshape(1, _ENT)
    tb1 = tv_b1.reshape(1, 128)
    tb2 = tv_b2.reshape(1, 128)
    out_a = _tc_dense(emb_a, bias_blk, hp2d[:half], hpw_p, uw_p, ub,
                      tv_w1, tb1, tv_w2, tb2, batch // 2)
    out_b = _tc_dense(emb_b, bias_blk, hp2d[half:], hpw_p, uw_p, ub,
                      tv_w1, tb1, tv_w2, tb2, batch // 2)
    return jnp.concatenate([out_a, out_b])
